# Initial kernel scaffold; baseline (speedup 1.0000x reference)
import jax
import jax.numpy as jnp
from jax.experimental import pallas as pl


def kernel(x, edge_index, edge_attr, W1, b1, W2, b2, fc1_w, fc1_b, fc2_w, fc2_b):
    raise NotImplementedError("write your pallas kernel here")


# speedup vs baseline: 15.6495x; 15.6495x over previous
import functools

import jax
import jax.numpy as jnp
from jax import lax
from jax.experimental import pallas as pl
from jax.experimental.pallas import tpu as pltpu
from jax.experimental.pallas import tpu_sc as plsc

N = 10000
E = 320000
D_IN = 128
DH = 64
OUT = 10
PN = 10240
NC = 2
NS = 16
NW = NC * NS
L = 16
EPW = E // NW
CH = 128
NCH = E // CH
KF = NCH // NW
KR = NCH - KF * NW
KMAX = KF + 1
NBUF = 4

_mesh = plsc.VectorSubcoreMesh(
    core_axis_name="c", subcore_axis_name="s", num_cores=NC, num_subcores=NS)

_f32 = jnp.float32
_i32 = jnp.int32
_sc_params = pltpu.CompilerParams(
    needs_layout_passes=False, use_tc_tiling_on_sc=False)


def _zero_1d(ref, n):
    z = jnp.zeros((L,), _f32)

    def body(i, c):
        ref[pl.ds(i * L, L)] = z
        return c

    lax.fori_loop(0, n // L, body, 0)


@functools.partial(
    pl.kernel,
    out_type=jax.ShapeDtypeStruct((NW, PN), _f32),
    mesh=_mesh,
    compiler_params=_sc_params,
    scratch_types=[
        pltpu.VMEM((EPW,), _i32),
        pltpu.VMEM((EPW, 4), _f32),
        pltpu.VMEM((PN,), _f32),
    ],
)
def _deg_kernel(ei_hbm, ea_hbm, out_hbm, didx_v, w4_v, acc_v):
    c = lax.axis_index("c")
    s = lax.axis_index("s")
    wid = c * NS + s
    _zero_1d(acc_v, PN)
    base = wid * EPW
    pltpu.sync_copy(ei_hbm.at[1, pl.ds(base, EPW)], didx_v)
    pltpu.sync_copy(ea_hbm.at[pl.ds(base, EPW)], w4_v)
    iota = lax.iota(_i32, L)
    zci = jnp.zeros((L,), _i32)

    def body(i, carry):
        idx = didx_v[pl.ds(i * L, L)]
        w = plsc.load_gather(w4_v, [i * L + iota, zci])
        plsc.addupdate_scatter(acc_v, [idx], w)
        return carry

    lax.fori_loop(0, EPW // L, body, 0)
    pltpu.sync_copy(acc_v, out_hbm.at[wid])


@functools.partial(
    pl.kernel,
    out_type=jax.ShapeDtypeStruct((NC, PN, DH), _f32),
    mesh=_mesh,
    compiler_params=_sc_params,
    scratch_types=[
        pltpu.VMEM((KMAX, CH), _i32),
        pltpu.VMEM((KMAX, CH), _i32),
        pltpu.VMEM((NBUF, CH, 4), _f32),
        pltpu.VMEM((NBUF, CH, DH), _f32),
        pltpu.VMEM_SHARED((PN, DH), _f32),
        pltpu.SemaphoreType.DMA,
        pltpu.SemaphoreType.DMA,
        pltpu.SemaphoreType.DMA,
        pltpu.SemaphoreType.DMA,
        pltpu.SemaphoreType.DMA,
        pltpu.SemaphoreType.DMA,
        pltpu.SemaphoreType.DMA,
        pltpu.SemaphoreType.DMA,
        pltpu.SemaphoreType.DMA,
    ],
)
def _l1_kernel(y_hbm, ei3_hbm, ea3_hbm, out_hbm,
               sidxB, didxB, w4_v, rows_v, acc_sh,
               sp, g0, g1, g2, g3, s0, s1, s2, s3):
    c = lax.axis_index("c")
    s = lax.axis_index("s")
    wid = c * NS + s
    semg = (g0, g1, g2, g3)
    sems = (s0, s1, s2, s3)
    nch = KF + jnp.where(wid < KR, 1, 0)
    c0 = KF * wid + jnp.minimum(wid, KR)
    cp1 = pltpu.async_copy(ei3_hbm.at[0, pl.ds(c0, KF)],
                           sidxB.at[pl.ds(0, KF)], sp)
    cp2 = pltpu.async_copy(ei3_hbm.at[1, pl.ds(c0, KF)],
                           didxB.at[pl.ds(0, KF)], sp)

    @pl.when(wid < KR)
    def _():
        pltpu.async_copy(ei3_hbm.at[0, pl.ds(c0 + KF, 1)],
                         sidxB.at[pl.ds(KF, 1)], sp)
        pltpu.async_copy(ei3_hbm.at[1, pl.ds(c0 + KF, 1)],
                         didxB.at[pl.ds(KF, 1)], sp)

    zrow = jnp.zeros((L,), _f32)

    def zr(r, carry):
        for j in range(DH // L):
            rows_v[0, r, pl.ds(j * L, L)] = zrow
        return carry

    lax.fori_loop(0, CH, zr, 0)
    rows_per_tile = PN // NS

    def zc(k, carry):
        pltpu.sync_copy(rows_v.at[0],
                        acc_sh.at[pl.ds(s * rows_per_tile + k * CH, CH)])
        return carry

    lax.fori_loop(0, rows_per_tile // CH, zc, 0)
    plsc.subcore_barrier()
    cp1.wait()
    cp2.wait()

    @pl.when(wid < KR)
    def _():
        pltpu.make_async_copy(ei3_hbm.at[0, pl.ds(0, 1)],
                              sidxB.at[pl.ds(KF, 1)], sp).wait()
        pltpu.make_async_copy(ei3_hbm.at[1, pl.ds(0, 1)],
                              didxB.at[pl.ds(KF, 1)], sp).wait()

    zci = jnp.zeros((L,), _i32)

    def fire_chunk(k, b):
        pltpu.async_copy(y_hbm.at[sidxB.at[k]], rows_v.at[b], semg[b])
        pltpu.async_copy(ea3_hbm.at[c0 + k], w4_v.at[b], semg[b])

    def wait_chunk(k, b):
        pltpu.make_async_copy(y_hbm.at[sidxB.at[k]], rows_v.at[b],
                              semg[b]).wait()
        pltpu.make_async_copy(ea3_hbm.at[0], w4_v.at[b], semg[b]).wait()

    def fire_scatter(k, b):
        pltpu.async_copy(rows_v.at[b], acc_sh.at[didxB.at[k]], sems[b],
                         add=True)

    def wait_scatter(k, b):
        pltpu.make_async_copy(rows_v.at[b], acc_sh.at[didxB.at[k]],
                              sems[b]).wait()

    fire_chunk(0, 0)
    fire_chunk(1, 1)

    def slot(k, b):
        b2 = (b + 2) % NBUF

        @pl.when(k < nch)
        def _():
            wait_chunk(k, b)

            @plsc.parallel_loop(0, CH, 1, unroll=4)
            def _(r):
                w = plsc.load_gather(w4_v.at[b], [jnp.full((L,), r, _i32),
                                                  zci])
                for j in range(DH // L):
                    rows_v[b, r, pl.ds(j * L, L)] = (
                        rows_v[b, r, pl.ds(j * L, L)] * w)

            fire_scatter(k, b)

            @pl.when(k + 2 < nch)
            def _():
                @pl.when(k >= 2)
                def _():
                    wait_scatter(k - 2, b2)

                fire_chunk(k + 2, b2)

    def body(q, carry):
        for b in range(NBUF):
            slot(q * NBUF + b, b)
        return carry

    lax.fori_loop(0, KMAX // NBUF + 1, body, 0)

    @pl.when(wid < KR)
    def _():
        for j in range(KMAX - 4, KMAX):
            wait_scatter(j, j % NBUF)

    @pl.when(wid >= KR)
    def _():
        for j in range(KF - 4, KF):
            wait_scatter(j, j % NBUF)

    plsc.subcore_barrier()

    def xb(k, carry):
        r0 = s * rows_per_tile + k * CH
        pltpu.sync_copy(acc_sh.at[pl.ds(r0, CH)], out_hbm.at[c, pl.ds(r0, CH)])
        return carry

    lax.fori_loop(0, rows_per_tile // CH, xb, 0)


@functools.partial(
    pl.kernel,
    out_type=jax.ShapeDtypeStruct((NW, PN), _f32),
    mesh=_mesh,
    compiler_params=_sc_params,
    scratch_types=[
        pltpu.VMEM((EPW,), _i32),
        pltpu.VMEM((EPW,), _i32),
        pltpu.VMEM((EPW, 4), _f32),
        pltpu.VMEM((N,), _f32),
        pltpu.VMEM((PN,), _f32),
    ],
)
def _l2_kernel(ei_hbm, ea_hbm, y2_hbm, out_hbm,
               sidx_v, didx_v, w4_v, tab_v, acc_v):
    c = lax.axis_index("c")
    s = lax.axis_index("s")
    wid = c * NS + s
    _zero_1d(acc_v, PN)
    base = wid * EPW
    pltpu.sync_copy(ei_hbm.at[0, pl.ds(base, EPW)], sidx_v)
    pltpu.sync_copy(ei_hbm.at[1, pl.ds(base, EPW)], didx_v)
    pltpu.sync_copy(ea_hbm.at[pl.ds(base, EPW)], w4_v)
    pltpu.sync_copy(y2_hbm, tab_v)
    iota = lax.iota(_i32, L)
    zci = jnp.zeros((L,), _i32)

    def body(i, carry):
        s16 = sidx_v[pl.ds(i * L, L)]
        d16 = didx_v[pl.ds(i * L, L)]
        w16 = plsc.load_gather(w4_v, [i * L + iota, zci])
        vals = plsc.load_gather(tab_v, [s16])
        plsc.addupdate_scatter(acc_v, [d16], w16 * vals)
        return carry

    lax.fori_loop(0, EPW // L, body, 0)
    pltpu.sync_copy(acc_v, out_hbm.at[wid])


def _lrelu(z):
    return jnp.where(z >= 0, z, 0.01 * z)


def _prep_body(degp_ref, x_ref, w1_ref, y_ref, dinv_ref):
    deg = 1.0 + jnp.sum(degp_ref[...], axis=1, keepdims=True)
    dinv = jnp.where(deg > 0, lax.rsqrt(deg), 0.0)
    d = dinv[:N]
    xw = jnp.dot(x_ref[...], w1_ref[...], preferred_element_type=_f32)
    y_ref[...] = xw * d
    dinv_ref[...] = d


def _mid_body(accp_ref, y_ref, dinv_ref, b1_ref, w2_ref, y2_ref):
    acc = accp_ref[0, :N, :] + accp_ref[1, :N, :]
    d = dinv_ref[...]
    h1 = _lrelu(d * (acc + y_ref[...]) + b1_ref[...])
    hw = jnp.dot(h1, w2_ref[...], preferred_element_type=_f32)
    y2_ref[...] = d * hw


def _fin_body(acc2p_ref, y2_ref, dinv_ref, b2_ref, fc1w_ref, fc1b_ref,
              fc2w_ref, fc2b_ref, out_ref):
    a2 = jnp.sum(acc2p_ref[...], axis=1, keepdims=True)
    h2 = _lrelu(dinv_ref[...] * (a2 + y2_ref[...]) + b2_ref[...])
    t = jnp.dot(fc1w_ref[...], h2, preferred_element_type=_f32)
    t = _lrelu(t + fc1b_ref[...])
    logits = jnp.dot(fc2w_ref[...], t, preferred_element_type=_f32)
    logits = logits + fc2b_ref[...]
    m = jnp.max(logits, axis=0, keepdims=True)
    e = jnp.exp(logits - m)
    out_ref[...] = e / jnp.sum(e, axis=0, keepdims=True)


def kernel(x, edge_index, edge_attr, W1, b1, W2, b2, fc1_w, fc1_b, fc2_w, fc2_b):
    ei3 = edge_index.reshape(2, NCH, CH)
    ea3 = edge_attr.reshape(NCH, CH, 4)

    degp = _deg_kernel(edge_index, edge_attr)
    y, dinv = pl.pallas_call(
        _prep_body,
        out_shape=[
            jax.ShapeDtypeStruct((N, DH), _f32),
            jax.ShapeDtypeStruct((N, 1), _f32),
        ],
    )(degp.T, x, W1)

    accp = _l1_kernel(y, ei3, ea3)
    y2 = pl.pallas_call(
        _mid_body,
        out_shape=jax.ShapeDtypeStruct((N, 1), _f32),
    )(accp, y, dinv, b1.reshape(1, DH), W2)

    acc2p = _l2_kernel(edge_index, edge_attr, y2.reshape(N))
    out = pl.pallas_call(
        _fin_body,
        out_shape=jax.ShapeDtypeStruct((OUT, 1), _f32),
    )(acc2p[:, :N].T, y2, dinv, b2.reshape(1, 1), fc1_w, fc1_b.reshape(D_IN, 1),
      fc2_w, fc2_b.reshape(OUT, 1))
    return out.reshape(1, OUT)


# speedup vs baseline: 27.2650x; 1.7422x over previous
import functools

import jax
import jax.numpy as jnp
from jax import lax
from jax.experimental import pallas as pl
from jax.experimental.pallas import tpu as pltpu
from jax.experimental.pallas import tpu_sc as plsc

N = 10000
E = 320000
D_IN = 128
DH = 64
OUT = 10
PN = 10240
NC = 2
NS = 16
NW = NC * NS
L = 16
EPW = E // NW
CH = 128
NCH = E // CH
KF = NCH // NW
KR = NCH - KF * NW
KMAX = KF + 1
NBUF = 4

_mesh = plsc.VectorSubcoreMesh(
    core_axis_name="c", subcore_axis_name="s", num_cores=NC, num_subcores=NS)

_f32 = jnp.float32
_i32 = jnp.int32
_sc_params = pltpu.CompilerParams(
    needs_layout_passes=False, use_tc_tiling_on_sc=False)


def _zero_1d(ref, n):
    z = jnp.zeros((L,), _f32)

    def body(i, c):
        ref[pl.ds(i * L, L)] = z
        return c

    lax.fori_loop(0, n // L, body, 0)


@functools.partial(
    pl.kernel,
    out_type=jax.ShapeDtypeStruct((NW, PN), _f32),
    mesh=_mesh,
    compiler_params=_sc_params,
    scratch_types=[
        pltpu.VMEM((EPW,), _i32),
        pltpu.VMEM((EPW * 4,), _f32),
        pltpu.VMEM((PN,), _f32),
    ],
)
def _deg_kernel(ei_hbm, ea_hbm, out_hbm, didx_v, w4_v, acc_v):
    c = lax.axis_index("c")
    s = lax.axis_index("s")
    wid = c * NS + s
    _zero_1d(acc_v, PN)
    base = wid * EPW
    pltpu.sync_copy(ei_hbm.at[pl.ds(E + base, EPW)], didx_v)
    pltpu.sync_copy(ea_hbm.at[pl.ds(base * 4, EPW * 4)], w4_v)
    iota4 = lax.iota(_i32, L) * 4

    def body(i, carry):
        idx = didx_v[pl.ds(i * L, L)]
        w = plsc.load_gather(w4_v, [i * (L * 4) + iota4])
        plsc.addupdate_scatter(acc_v, [idx], w)
        return carry

    lax.fori_loop(0, EPW // L, body, 0)
    pltpu.sync_copy(acc_v, out_hbm.at[wid])


@functools.partial(
    pl.kernel,
    out_type=jax.ShapeDtypeStruct((NC, PN, DH), _f32),
    mesh=_mesh,
    compiler_params=_sc_params,
    scratch_types=[
        pltpu.VMEM((KMAX, CH), _i32),
        pltpu.VMEM((KMAX, CH), _i32),
        pltpu.VMEM((NBUF, CH * 4), _f32),
        pltpu.VMEM((NBUF, CH, DH), _f32),
        pltpu.VMEM_SHARED((PN, DH), _f32),
        pltpu.SemaphoreType.DMA,
        pltpu.SemaphoreType.DMA,
        pltpu.SemaphoreType.DMA,
        pltpu.SemaphoreType.DMA,
        pltpu.SemaphoreType.DMA,
        pltpu.SemaphoreType.DMA,
        pltpu.SemaphoreType.DMA,
        pltpu.SemaphoreType.DMA,
        pltpu.SemaphoreType.DMA,
    ],
)
def _l1_kernel(y_hbm, ei3_hbm, ea_hbm, out_hbm,
               sidxB, didxB, w4_v, rows_v, acc_sh,
               sp, g0, g1, g2, g3, s0, s1, s2, s3):
    c = lax.axis_index("c")
    s = lax.axis_index("s")
    wid = c * NS + s
    semg = (g0, g1, g2, g3)
    sems = (s0, s1, s2, s3)
    nch = KF + jnp.where(wid < KR, 1, 0)
    c0 = KF * wid + jnp.minimum(wid, KR)
    cp1 = pltpu.async_copy(ei3_hbm.at[0, pl.ds(c0, KF)],
                           sidxB.at[pl.ds(0, KF)], sp)
    cp2 = pltpu.async_copy(ei3_hbm.at[1, pl.ds(c0, KF)],
                           didxB.at[pl.ds(0, KF)], sp)

    @pl.when(wid < KR)
    def _():
        pltpu.async_copy(ei3_hbm.at[0, pl.ds(c0 + KF, 1)],
                         sidxB.at[pl.ds(KF, 1)], sp)
        pltpu.async_copy(ei3_hbm.at[1, pl.ds(c0 + KF, 1)],
                         didxB.at[pl.ds(KF, 1)], sp)

    zrow = jnp.zeros((L,), _f32)

    def zr(r, carry):
        for j in range(DH // L):
            rows_v[0, r, pl.ds(j * L, L)] = zrow
        return carry

    lax.fori_loop(0, CH, zr, 0)
    rows_per_tile = PN // NS

    def zc(k, carry):
        pltpu.sync_copy(rows_v.at[0],
                        acc_sh.at[pl.ds(s * rows_per_tile + k * CH, CH)])
        return carry

    lax.fori_loop(0, rows_per_tile // CH, zc, 0)
    plsc.subcore_barrier()
    cp1.wait()
    cp2.wait()

    @pl.when(wid < KR)
    def _():
        pltpu.make_async_copy(ei3_hbm.at[0, pl.ds(0, 1)],
                              sidxB.at[pl.ds(KF, 1)], sp).wait()
        pltpu.make_async_copy(ei3_hbm.at[1, pl.ds(0, 1)],
                              didxB.at[pl.ds(KF, 1)], sp).wait()

    def fire_chunk(k, b):
        pltpu.async_copy(y_hbm.at[sidxB.at[k]], rows_v.at[b], semg[b])
        pltpu.async_copy(ea_hbm.at[pl.ds((c0 + k) * (CH * 4), CH * 4)],
                         w4_v.at[b], semg[b])

    def wait_chunk(k, b):
        pltpu.make_async_copy(y_hbm.at[sidxB.at[k]], rows_v.at[b],
                              semg[b]).wait()
        pltpu.make_async_copy(ea_hbm.at[pl.ds(0, CH * 4)], w4_v.at[b],
                              semg[b]).wait()

    def fire_scatter(k, b):
        pltpu.async_copy(rows_v.at[b], acc_sh.at[didxB.at[k]], sems[b],
                         add=True)

    def wait_scatter(k, b):
        pltpu.make_async_copy(rows_v.at[b], acc_sh.at[didxB.at[k]],
                              sems[b]).wait()

    fire_chunk(0, 0)
    fire_chunk(1, 1)

    def slot(k, b):
        b2 = (b + 2) % NBUF

        @pl.when(k < nch)
        def _():
            wait_chunk(k, b)

            @plsc.parallel_loop(0, CH, 1, unroll=4)
            def _(r):
                w = plsc.load_gather(w4_v.at[b], [jnp.full((L,), r * 4, _i32)])
                for j in range(DH // L):
                    rows_v[b, r, pl.ds(j * L, L)] = (
                        rows_v[b, r, pl.ds(j * L, L)] * w)

            fire_scatter(k, b)

            @pl.when(k + 2 < nch)
            def _():
                @pl.when(k >= 2)
                def _():
                    wait_scatter(k - 2, b2)

                fire_chunk(k + 2, b2)

    def body(q, carry):
        for b in range(NBUF):
            slot(q * NBUF + b, b)
        return carry

    lax.fori_loop(0, KMAX // NBUF + 1, body, 0)

    @pl.when(wid < KR)
    def _():
        for j in range(KMAX - 4, KMAX):
            wait_scatter(j, j % NBUF)

    @pl.when(wid >= KR)
    def _():
        for j in range(KF - 4, KF):
            wait_scatter(j, j % NBUF)

    plsc.subcore_barrier()

    def xb(k, carry):
        r0 = s * rows_per_tile + k * CH
        pltpu.sync_copy(acc_sh.at[pl.ds(r0, CH)], out_hbm.at[c, pl.ds(r0, CH)])
        return carry

    lax.fori_loop(0, rows_per_tile // CH, xb, 0)


@functools.partial(
    pl.kernel,
    out_type=jax.ShapeDtypeStruct((NW, PN), _f32),
    mesh=_mesh,
    compiler_params=_sc_params,
    scratch_types=[
        pltpu.VMEM((EPW,), _i32),
        pltpu.VMEM((EPW,), _i32),
        pltpu.VMEM((EPW * 4,), _f32),
        pltpu.VMEM((N,), _f32),
        pltpu.VMEM((PN,), _f32),
    ],
)
def _l2_kernel(ei_hbm, ea_hbm, y2_hbm, out_hbm,
               sidx_v, didx_v, w4_v, tab_v, acc_v):
    c = lax.axis_index("c")
    s = lax.axis_index("s")
    wid = c * NS + s
    _zero_1d(acc_v, PN)
    base = wid * EPW
    pltpu.sync_copy(ei_hbm.at[pl.ds(base, EPW)], sidx_v)
    pltpu.sync_copy(ei_hbm.at[pl.ds(E + base, EPW)], didx_v)
    pltpu.sync_copy(ea_hbm.at[pl.ds(base * 4, EPW * 4)], w4_v)
    pltpu.sync_copy(y2_hbm, tab_v)
    iota4 = lax.iota(_i32, L) * 4

    def body(i, carry):
        s16 = sidx_v[pl.ds(i * L, L)]
        d16 = didx_v[pl.ds(i * L, L)]
        w16 = plsc.load_gather(w4_v, [i * (L * 4) + iota4])
        vals = plsc.load_gather(tab_v, [s16])
        plsc.addupdate_scatter(acc_v, [d16], w16 * vals)
        return carry

    lax.fori_loop(0, EPW // L, body, 0)
    pltpu.sync_copy(acc_v, out_hbm.at[wid])


def _lrelu(z):
    return jnp.where(z >= 0, z, 0.01 * z)


def _prep_body(degp_ref, x_ref, w1_ref, y_ref, dinv_ref):
    deg = 1.0 + jnp.sum(degp_ref[...], axis=1, keepdims=True)
    dinv = jnp.where(deg > 0, lax.rsqrt(deg), 0.0)
    d = dinv[:N]
    xw = jnp.dot(x_ref[...], w1_ref[...], preferred_element_type=_f32)
    y_ref[...] = xw * d
    dinv_ref[...] = d


def _mid_body(accp_ref, y_ref, dinv_ref, b1_ref, w2_ref, y2_ref):
    acc = accp_ref[0, :N, :] + accp_ref[1, :N, :]
    d = dinv_ref[...]
    h1 = _lrelu(d * (acc + y_ref[...]) + b1_ref[...])
    hw = jnp.dot(h1, w2_ref[...], preferred_element_type=_f32)
    y2_ref[...] = d * hw


def _fin_body(acc2p_ref, y2_ref, dinv_ref, b2_ref, fc1w_ref, fc1b_ref,
              fc2w_ref, fc2b_ref, out_ref):
    a2 = jnp.sum(acc2p_ref[...], axis=1, keepdims=True)
    h2 = _lrelu(dinv_ref[...] * (a2 + y2_ref[...]) + b2_ref[...])
    t = jnp.dot(fc1w_ref[...], h2, preferred_element_type=_f32)
    t = _lrelu(t + fc1b_ref[...])
    logits = jnp.dot(fc2w_ref[...], t, preferred_element_type=_f32)
    logits = logits + fc2b_ref[...]
    m = jnp.max(logits, axis=0, keepdims=True)
    e = jnp.exp(logits - m)
    out_ref[...] = e / jnp.sum(e, axis=0, keepdims=True)


def kernel(x, edge_index, edge_attr, W1, b1, W2, b2, fc1_w, fc1_b, fc2_w, fc2_b):
    ei3 = edge_index.reshape(2, NCH, CH)
    ei_flat = edge_index.reshape(2 * E)
    ea_flat = edge_attr.reshape(4 * E)

    degp = _deg_kernel(ei_flat, ea_flat)
    y, dinv = pl.pallas_call(
        _prep_body,
        out_shape=[
            jax.ShapeDtypeStruct((N, DH), _f32),
            jax.ShapeDtypeStruct((N, 1), _f32),
        ],
    )(degp.T, x, W1)

    accp = _l1_kernel(y, ei3, ea_flat)
    y2 = pl.pallas_call(
        _mid_body,
        out_shape=jax.ShapeDtypeStruct((N, 1), _f32),
    )(accp, y, dinv, b1.reshape(1, DH), W2)

    acc2p = _l2_kernel(ei_flat, ea_flat, y2.reshape(N))
    out = pl.pallas_call(
        _fin_body,
        out_shape=jax.ShapeDtypeStruct((OUT, 1), _f32),
    )(acc2p[:, :N].T, y2, dinv, b2.reshape(1, 1), fc1_w, fc1_b.reshape(D_IN, 1),
      fc2_w, fc2_b.reshape(OUT, 1))
    return out.reshape(1, OUT)


# speedup vs baseline: 50.5929x; 1.8556x over previous
import functools

import jax
import jax.numpy as jnp
from jax import lax
from jax.experimental import pallas as pl
from jax.experimental.pallas import tpu as pltpu
from jax.experimental.pallas import tpu_sc as plsc

N = 10000
E = 320000
D_IN = 128
DH = 64
OUT = 10
PN = 10240
NC = 2
NS = 16
NW = NC * NS
L = 16
EPW = E // NW
CH = 128
KCH = 80
G = 2
SK = KCH // G
EP = NW * KCH * CH

_mesh = plsc.VectorSubcoreMesh(
    core_axis_name="c", subcore_axis_name="s", num_cores=NC, num_subcores=NS)

_f32 = jnp.float32
_i32 = jnp.int32
_sc_params = pltpu.CompilerParams(
    needs_layout_passes=False, use_tc_tiling_on_sc=False)


def _zero_1d(ref, n):
    z = jnp.zeros((L,), _f32)

    def body(i, c):
        ref[pl.ds(i * L, L)] = z
        return c

    lax.fori_loop(0, n // L, body, 0)


@functools.partial(
    pl.kernel,
    out_type=jax.ShapeDtypeStruct((NW, PN), _f32),
    mesh=_mesh,
    compiler_params=_sc_params,
    scratch_types=[
        pltpu.VMEM((EPW,), _i32),
        pltpu.VMEM((EPW,), _f32),
        pltpu.VMEM((PN,), _f32),
    ],
)
def _deg_kernel(dst_hbm, ew_hbm, out_hbm, didx_v, w_v, acc_v):
    c = lax.axis_index("c")
    s = lax.axis_index("s")
    wid = c * NS + s
    _zero_1d(acc_v, PN)
    base = wid * EPW
    pltpu.sync_copy(dst_hbm.at[pl.ds(base, EPW)], didx_v)
    pltpu.sync_copy(ew_hbm.at[pl.ds(base, EPW)], w_v)

    def body(i, carry):
        idx = didx_v[pl.ds(i * L, L)]
        w = w_v[pl.ds(i * L, L)]
        plsc.addupdate_scatter(acc_v, [idx], w)
        return carry

    lax.fori_loop(0, EPW // L, body, 0)
    pltpu.sync_copy(acc_v, out_hbm.at[wid])


@functools.partial(
    pl.kernel,
    out_type=jax.ShapeDtypeStruct((NC, PN, DH), _f32),
    mesh=_mesh,
    compiler_params=_sc_params,
    scratch_types=[
        pltpu.VMEM((KCH, CH), _i32),
        pltpu.VMEM((KCH, CH), _i32),
        pltpu.VMEM((KCH, CH), _f32),
        pltpu.VMEM((2, G, CH, DH), _f32),
        pltpu.VMEM_SHARED((PN, DH), _f32),
        pltpu.SemaphoreType.DMA,
        pltpu.SemaphoreType.DMA,
        pltpu.SemaphoreType.DMA,
        pltpu.SemaphoreType.DMA,
    ],
)
def _l1_kernel(y_hbm, src_hbm, dst_hbm, ew_hbm, out_hbm,
               sidxB, didxB, wB, rows_v, acc_sh, sg0, sg1, ss0, ss1):
    c = lax.axis_index("c")
    s = lax.axis_index("s")
    wid = c * NS + s
    semg = (sg0, sg1)
    sems = (ss0, ss1)
    rb = wid * KCH
    cp_s = pltpu.async_copy(src_hbm.at[pl.ds(rb, KCH)], sidxB, sg0)
    cp_d = pltpu.async_copy(dst_hbm.at[pl.ds(rb, KCH)], didxB, sg0)
    cp_w = pltpu.async_copy(ew_hbm.at[pl.ds(rb, KCH)], wB, sg0)
    zrow = jnp.zeros((L,), _f32)

    def zr(r, carry):
        for j in range(DH // L):
            rows_v[0, 0, r, pl.ds(j * L, L)] = zrow
        return carry

    lax.fori_loop(0, CH, zr, 0)
    rows_per_tile = PN // NS

    def zc(k, carry):
        pltpu.sync_copy(rows_v.at[0, 0],
                        acc_sh.at[pl.ds(s * rows_per_tile + k * CH, CH)])
        return carry

    lax.fori_loop(0, rows_per_tile // CH, zc, 0)
    plsc.subcore_barrier()
    cp_s.wait()
    cp_d.wait()
    cp_w.wait()

    def fire_gather(sk, b):
        for g in range(G):
            pltpu.async_copy(y_hbm.at[sidxB.at[sk * G + g]],
                             rows_v.at[b, g], semg[b])

    def wait_gather(sk, b):
        for g in range(G):
            pltpu.make_async_copy(y_hbm.at[sidxB.at[sk * G + g]],
                                  rows_v.at[b, g], semg[b]).wait()

    def fire_scatter(sk, b):
        for g in range(G):
            pltpu.async_copy(rows_v.at[b, g], acc_sh.at[didxB.at[sk * G + g]],
                             sems[b], add=True)

    def wait_scatter(sk, b):
        for g in range(G):
            pltpu.make_async_copy(rows_v.at[b, g],
                                  acc_sh.at[didxB.at[sk * G + g]],
                                  sems[b]).wait()

    fire_gather(0, 0)

    def half(sk, b):
        nb = 1 - b

        @pl.when(sk > 0)
        def _():
            wait_scatter(sk - 1, nb)

        @pl.when(sk + 1 < SK)
        def _():
            fire_gather(sk + 1, nb)

        wait_gather(sk, b)

        @plsc.parallel_loop(0, CH, 1, unroll=4)
        def _(r):
            ridx = jnp.full((L,), r, _i32)
            for g in range(G):
                w = plsc.load_gather(wB.at[sk * G + g], [ridx])
                for j in range(DH // L):
                    rows_v[b, g, r, pl.ds(j * L, L)] = (
                        rows_v[b, g, r, pl.ds(j * L, L)] * w)

        fire_scatter(sk, b)

    def body(k2, carry):
        half(2 * k2, 0)
        half(2 * k2 + 1, 1)
        return carry

    lax.fori_loop(0, SK // 2, body, 0)
    wait_scatter(SK - 1, 1)
    plsc.subcore_barrier()

    def xb(k, carry):
        r0 = s * rows_per_tile + k * CH
        pltpu.sync_copy(acc_sh.at[pl.ds(r0, CH)], out_hbm.at[c, pl.ds(r0, CH)])
        return carry

    lax.fori_loop(0, rows_per_tile // CH, xb, 0)


@functools.partial(
    pl.kernel,
    out_type=jax.ShapeDtypeStruct((NW, PN), _f32),
    mesh=_mesh,
    compiler_params=_sc_params,
    scratch_types=[
        pltpu.VMEM((EPW,), _i32),
        pltpu.VMEM((EPW,), _i32),
        pltpu.VMEM((EPW,), _f32),
        pltpu.VMEM((N,), _f32),
        pltpu.VMEM((PN,), _f32),
    ],
)
def _l2_kernel(src_hbm, dst_hbm, ew_hbm, y2_hbm, out_hbm,
               sidx_v, didx_v, w_v, tab_v, acc_v):
    c = lax.axis_index("c")
    s = lax.axis_index("s")
    wid = c * NS + s
    _zero_1d(acc_v, PN)
    base = wid * EPW
    pltpu.sync_copy(src_hbm.at[pl.ds(base, EPW)], sidx_v)
    pltpu.sync_copy(dst_hbm.at[pl.ds(base, EPW)], didx_v)
    pltpu.sync_copy(ew_hbm.at[pl.ds(base, EPW)], w_v)
    pltpu.sync_copy(y2_hbm, tab_v)

    def body(i, carry):
        s16 = sidx_v[pl.ds(i * L, L)]
        d16 = didx_v[pl.ds(i * L, L)]
        w16 = w_v[pl.ds(i * L, L)]
        vals = plsc.load_gather(tab_v, [s16])
        plsc.addupdate_scatter(acc_v, [d16], w16 * vals)
        return carry

    lax.fori_loop(0, EPW // L, body, 0)
    pltpu.sync_copy(acc_v, out_hbm.at[wid])


def _lrelu(z):
    return jnp.where(z >= 0, z, 0.01 * z)


def _prep_body(degp_ref, x_ref, w1_ref, y_ref, dinv_ref):
    deg = 1.0 + jnp.sum(degp_ref[...], axis=1, keepdims=True)
    dinv = jnp.where(deg > 0, lax.rsqrt(deg), 0.0)
    d = dinv[:N]
    xw = jnp.dot(x_ref[...], w1_ref[...], preferred_element_type=_f32)
    y_ref[...] = xw * d
    dinv_ref[...] = d


def _mid_body(accp_ref, y_ref, dinv_ref, b1_ref, w2_ref, y2_ref):
    acc = accp_ref[0, :N, :] + accp_ref[1, :N, :]
    d = dinv_ref[...]
    h1 = _lrelu(d * (acc + y_ref[...]) + b1_ref[...])
    hw = jnp.dot(h1, w2_ref[...], preferred_element_type=_f32)
    y2_ref[...] = d * hw


def _fin_body(acc2p_ref, y2_ref, dinv_ref, b2_ref, fc1w_ref, fc1b_ref,
              fc2w_ref, fc2b_ref, out_ref):
    a2 = jnp.sum(acc2p_ref[...], axis=1, keepdims=True)
    h2 = _lrelu(dinv_ref[...] * (a2 + y2_ref[...]) + b2_ref[...])
    t = jnp.dot(fc1w_ref[...], h2, preferred_element_type=_f32)
    t = _lrelu(t + fc1b_ref[...])
    logits = jnp.dot(fc2w_ref[...], t, preferred_element_type=_f32)
    logits = logits + fc2b_ref[...]
    m = jnp.max(logits, axis=0, keepdims=True)
    e = jnp.exp(logits - m)
    out_ref[...] = e / jnp.sum(e, axis=0, keepdims=True)


def kernel(x, edge_index, edge_attr, W1, b1, W2, b2, fc1_w, fc1_b, fc2_w, fc2_b):
    src = edge_index[0]
    dst = edge_index[1]
    ew = edge_attr[:, 0]

    degp = _deg_kernel(dst, ew)
    y, dinv = pl.pallas_call(
        _prep_body,
        out_shape=[
            jax.ShapeDtypeStruct((N, DH), _f32),
            jax.ShapeDtypeStruct((N, 1), _f32),
        ],
    )(degp.T, x, W1)

    pad = EP - E
    pidx = jnp.arange(pad, dtype=jnp.int32)
    src_p = jnp.concatenate([src, pidx % N]).reshape(EP // CH, CH)
    dst_p = jnp.concatenate([dst, N + pidx % (PN - N)]).reshape(EP // CH, CH)
    ew_p = jnp.concatenate(
        [ew, jnp.zeros((pad,), _f32)]).reshape(EP // CH, CH)
    accp = _l1_kernel(y, src_p, dst_p, ew_p)
    y2 = pl.pallas_call(
        _mid_body,
        out_shape=jax.ShapeDtypeStruct((N, 1), _f32),
    )(accp, y, dinv, b1.reshape(1, DH), W2)

    acc2p = _l2_kernel(src, dst, ew, y2.reshape(N))
    out = pl.pallas_call(
        _fin_body,
        out_shape=jax.ShapeDtypeStruct((OUT, 1), _f32),
    )(acc2p[:, :N].T, y2, dinv, b2.reshape(1, 1), fc1_w, fc1_b.reshape(D_IN, 1),
      fc2_w, fc2_b.reshape(OUT, 1))
    return out.reshape(1, OUT)


# speedup vs baseline: 52.2445x; 1.0326x over previous
import functools

import jax
import jax.numpy as jnp
from jax import lax
from jax.experimental import pallas as pl
from jax.experimental.pallas import tpu as pltpu
from jax.experimental.pallas import tpu_sc as plsc

N = 10000
E = 320000
D_IN = 128
DH = 64
OUT = 10
PN = 10240
NC = 2
NS = 16
NW = NC * NS
L = 16
EPW = E // NW
CH = 128
KCH = 80
G = 2
SK = KCH // G
EP = NW * KCH * CH

_mesh = plsc.VectorSubcoreMesh(
    core_axis_name="c", subcore_axis_name="s", num_cores=NC, num_subcores=NS)

_f32 = jnp.float32
_i32 = jnp.int32
_sc_params = pltpu.CompilerParams(
    needs_layout_passes=False, use_tc_tiling_on_sc=False)


def _zero_1d(ref, n):
    z = jnp.zeros((L,), _f32)

    def body(i, c):
        ref[pl.ds(i * L, L)] = z
        return c

    lax.fori_loop(0, n // L, body, 0)


@functools.partial(
    pl.kernel,
    out_type=jax.ShapeDtypeStruct((NW, PN), _f32),
    mesh=_mesh,
    compiler_params=_sc_params,
    scratch_types=[
        pltpu.VMEM((EPW,), _i32),
        pltpu.VMEM((EPW,), _f32),
        pltpu.VMEM((PN,), _f32),
    ],
)
def _deg_kernel(dst_hbm, ew_hbm, out_hbm, didx_v, w_v, acc_v):
    c = lax.axis_index("c")
    s = lax.axis_index("s")
    wid = c * NS + s
    _zero_1d(acc_v, PN)
    base = wid * EPW
    pltpu.sync_copy(dst_hbm.at[pl.ds(base, EPW)], didx_v)
    pltpu.sync_copy(ew_hbm.at[pl.ds(base, EPW)], w_v)

    @plsc.parallel_loop(0, EPW // L, 1, unroll=8)
    def _(i):
        idx = didx_v[pl.ds(i * L, L)]
        w = w_v[pl.ds(i * L, L)]
        plsc.addupdate_scatter(acc_v, [idx], w)

    pltpu.sync_copy(acc_v, out_hbm.at[wid])


@functools.partial(
    pl.kernel,
    out_type=jax.ShapeDtypeStruct((NC, PN, DH), _f32),
    mesh=_mesh,
    compiler_params=_sc_params,
    scratch_types=[
        pltpu.VMEM((KCH, CH), _i32),
        pltpu.VMEM((KCH, CH), _i32),
        pltpu.VMEM((KCH, CH), _f32),
        pltpu.VMEM((2, G, CH, DH), _f32),
        pltpu.VMEM_SHARED((PN, DH), _f32),
        pltpu.SemaphoreType.DMA,
        pltpu.SemaphoreType.DMA,
        pltpu.SemaphoreType.DMA,
        pltpu.SemaphoreType.DMA,
    ],
)
def _l1_kernel(y_hbm, src_hbm, dst_hbm, ew_hbm, out_hbm,
               sidxB, didxB, wB, rows_v, acc_sh, sg0, sg1, ss0, ss1):
    c = lax.axis_index("c")
    s = lax.axis_index("s")
    wid = c * NS + s
    semg = (sg0, sg1)
    sems = (ss0, ss1)
    rb = wid * KCH
    cp_s = pltpu.async_copy(src_hbm.at[pl.ds(rb, KCH)], sidxB, sg0)
    cp_d = pltpu.async_copy(dst_hbm.at[pl.ds(rb, KCH)], didxB, sg0)
    cp_w = pltpu.async_copy(ew_hbm.at[pl.ds(rb, KCH)], wB, sg0)
    zrow = jnp.zeros((L,), _f32)

    def zr(r, carry):
        for j in range(DH // L):
            rows_v[0, 0, r, pl.ds(j * L, L)] = zrow
        return carry

    lax.fori_loop(0, CH, zr, 0)
    rows_per_tile = PN // NS

    def zc(k, carry):
        pltpu.sync_copy(rows_v.at[0, 0],
                        acc_sh.at[pl.ds(s * rows_per_tile + k * CH, CH)])
        return carry

    lax.fori_loop(0, rows_per_tile // CH, zc, 0)
    plsc.subcore_barrier()
    cp_s.wait()
    cp_d.wait()
    cp_w.wait()

    def fire_gather(sk, b):
        for g in range(G):
            pltpu.async_copy(y_hbm.at[sidxB.at[sk * G + g]],
                             rows_v.at[b, g], semg[b])

    def wait_gather(sk, b):
        for g in range(G):
            pltpu.make_async_copy(y_hbm.at[sidxB.at[sk * G + g]],
                                  rows_v.at[b, g], semg[b]).wait()

    def fire_scatter(sk, b):
        for g in range(G):
            pltpu.async_copy(rows_v.at[b, g], acc_sh.at[didxB.at[sk * G + g]],
                             sems[b], add=True)

    def wait_scatter(sk, b):
        for g in range(G):
            pltpu.make_async_copy(rows_v.at[b, g],
                                  acc_sh.at[didxB.at[sk * G + g]],
                                  sems[b]).wait()

    fire_gather(0, 0)

    def half(sk, b):
        nb = 1 - b

        @pl.when(sk > 0)
        def _():
            wait_scatter(sk - 1, nb)

        @pl.when(sk + 1 < SK)
        def _():
            fire_gather(sk + 1, nb)

        wait_gather(sk, b)

        @plsc.parallel_loop(0, CH, 1, unroll=4)
        def _(r):
            ridx = jnp.full((L,), r, _i32)
            for g in range(G):
                w = plsc.load_gather(wB.at[sk * G + g], [ridx])
                for j in range(DH // L):
                    rows_v[b, g, r, pl.ds(j * L, L)] = (
                        rows_v[b, g, r, pl.ds(j * L, L)] * w)

        fire_scatter(sk, b)

    def body(k2, carry):
        half(2 * k2, 0)
        half(2 * k2 + 1, 1)
        return carry

    lax.fori_loop(0, SK // 2, body, 0)
    wait_scatter(SK - 1, 1)
    plsc.subcore_barrier()

    def xb(k, carry):
        r0 = s * rows_per_tile + k * CH
        pltpu.sync_copy(acc_sh.at[pl.ds(r0, CH)], out_hbm.at[c, pl.ds(r0, CH)])
        return carry

    lax.fori_loop(0, rows_per_tile // CH, xb, 0)


@functools.partial(
    pl.kernel,
    out_type=jax.ShapeDtypeStruct((NW, PN), _f32),
    mesh=_mesh,
    compiler_params=_sc_params,
    scratch_types=[
        pltpu.VMEM((EPW,), _i32),
        pltpu.VMEM((EPW,), _i32),
        pltpu.VMEM((EPW,), _f32),
        pltpu.VMEM((N,), _f32),
        pltpu.VMEM((PN,), _f32),
    ],
)
def _l2_kernel(src_hbm, dst_hbm, ew_hbm, y2_hbm, out_hbm,
               sidx_v, didx_v, w_v, tab_v, acc_v):
    c = lax.axis_index("c")
    s = lax.axis_index("s")
    wid = c * NS + s
    _zero_1d(acc_v, PN)
    base = wid * EPW
    pltpu.sync_copy(src_hbm.at[pl.ds(base, EPW)], sidx_v)
    pltpu.sync_copy(dst_hbm.at[pl.ds(base, EPW)], didx_v)
    pltpu.sync_copy(ew_hbm.at[pl.ds(base, EPW)], w_v)
    pltpu.sync_copy(y2_hbm, tab_v)

    @plsc.parallel_loop(0, EPW // L, 1, unroll=8)
    def _(i):
        s16 = sidx_v[pl.ds(i * L, L)]
        d16 = didx_v[pl.ds(i * L, L)]
        w16 = w_v[pl.ds(i * L, L)]
        vals = plsc.load_gather(tab_v, [s16])
        plsc.addupdate_scatter(acc_v, [d16], w16 * vals)

    pltpu.sync_copy(acc_v, out_hbm.at[wid])


def _lrelu(z):
    return jnp.where(z >= 0, z, 0.01 * z)


def _xw_body(x_ref, w1_ref, xw_ref):
    xw_ref[...] = jnp.dot(x_ref[...], w1_ref[...],
                          preferred_element_type=_f32)


def _prep_body(degp_ref, xw_ref, y_ref, dinv_ref):
    deg = 1.0 + jnp.sum(degp_ref[...], axis=1, keepdims=True)
    dinv = jnp.where(deg > 0, lax.rsqrt(deg), 0.0)
    d = dinv[:N]
    y_ref[...] = xw_ref[...] * d
    dinv_ref[...] = d


def _mid_body(accp_ref, y_ref, dinv_ref, b1_ref, w2_ref, y2_ref):
    acc = accp_ref[0, :N, :] + accp_ref[1, :N, :]
    d = dinv_ref[...]
    h1 = _lrelu(d * (acc + y_ref[...]) + b1_ref[...])
    hw = jnp.dot(h1, w2_ref[...], preferred_element_type=_f32)
    y2_ref[...] = d * hw


def _fin_body(acc2p_ref, y2_ref, dinv_ref, b2_ref, fc1w_ref, fc1b_ref,
              fc2w_ref, fc2b_ref, out_ref):
    a2 = jnp.sum(acc2p_ref[...], axis=1, keepdims=True)
    h2 = _lrelu(dinv_ref[...] * (a2 + y2_ref[...]) + b2_ref[...])
    t = jnp.dot(fc1w_ref[...], h2, preferred_element_type=_f32)
    t = _lrelu(t + fc1b_ref[...])
    logits = jnp.dot(fc2w_ref[...], t, preferred_element_type=_f32)
    logits = logits + fc2b_ref[...]
    m = jnp.max(logits, axis=0, keepdims=True)
    e = jnp.exp(logits - m)
    out_ref[...] = e / jnp.sum(e, axis=0, keepdims=True)


def kernel(x, edge_index, edge_attr, W1, b1, W2, b2, fc1_w, fc1_b, fc2_w, fc2_b):
    src = edge_index[0]
    dst = edge_index[1]
    ew = edge_attr[:, 0]

    degp = _deg_kernel(dst, ew)
    xw = pl.pallas_call(
        _xw_body,
        out_shape=jax.ShapeDtypeStruct((N, DH), _f32),
    )(x, W1)
    y, dinv = pl.pallas_call(
        _prep_body,
        out_shape=[
            jax.ShapeDtypeStruct((N, DH), _f32),
            jax.ShapeDtypeStruct((N, 1), _f32),
        ],
    )(degp.T, xw)

    pad = EP - E
    pidx = jnp.arange(pad, dtype=jnp.int32)
    src_p = jnp.concatenate([src, pidx % N]).reshape(EP // CH, CH)
    dst_p = jnp.concatenate([dst, N + pidx % (PN - N)]).reshape(EP // CH, CH)
    ew_p = jnp.concatenate(
        [ew, jnp.zeros((pad,), _f32)]).reshape(EP // CH, CH)
    accp = _l1_kernel(y, src_p, dst_p, ew_p)
    y2 = pl.pallas_call(
        _mid_body,
        out_shape=jax.ShapeDtypeStruct((N, 1), _f32),
    )(accp, y, dinv, b1.reshape(1, DH), W2)

    acc2p = _l2_kernel(src, dst, ew, y2.reshape(N))
    out = pl.pallas_call(
        _fin_body,
        out_shape=jax.ShapeDtypeStruct((OUT, 1), _f32),
    )(acc2p[:, :N].T, y2, dinv, b2.reshape(1, 1), fc1_w, fc1_b.reshape(D_IN, 1),
      fc2_w, fc2_b.reshape(OUT, 1))
    return out.reshape(1, OUT)


# speedup vs baseline: 54.1455x; 1.0364x over previous
import functools

import jax
import jax.numpy as jnp
from jax import lax
from jax.experimental import pallas as pl
from jax.experimental.pallas import tpu as pltpu
from jax.experimental.pallas import tpu_sc as plsc

N = 10000
E = 320000
D_IN = 128
DH = 64
OUT = 10
PN = 10240
NC = 2
NS = 16
NW = NC * NS
L = 16
EPW = E // NW
CH = 128
KCH = 80
G = 2
SK = KCH // G
EP = NW * KCH * CH

_mesh = plsc.VectorSubcoreMesh(
    core_axis_name="c", subcore_axis_name="s", num_cores=NC, num_subcores=NS)

_f32 = jnp.float32
_i32 = jnp.int32
_sc_params = pltpu.CompilerParams(
    needs_layout_passes=False, use_tc_tiling_on_sc=False)


def _zero_1d(ref, n):
    z = jnp.zeros((L,), _f32)

    def body(i, c):
        ref[pl.ds(i * L, L)] = z
        return c

    lax.fori_loop(0, n // L, body, 0)


@functools.partial(
    pl.kernel,
    out_type=jax.ShapeDtypeStruct((NW, PN), _f32),
    mesh=_mesh,
    compiler_params=_sc_params,
    scratch_types=[
        pltpu.VMEM((EPW,), _i32),
        pltpu.VMEM((EPW,), _f32),
        pltpu.VMEM((PN,), _f32),
    ],
)
def _deg_kernel(dst_hbm, ew_hbm, out_hbm, didx_v, w_v, acc_v):
    c = lax.axis_index("c")
    s = lax.axis_index("s")
    wid = c * NS + s
    _zero_1d(acc_v, PN)
    base = wid * EPW
    pltpu.sync_copy(dst_hbm.at[pl.ds(base, EPW)], didx_v)
    pltpu.sync_copy(ew_hbm.at[pl.ds(base, EPW)], w_v)

    @plsc.parallel_loop(0, EPW // L, 1, unroll=8)
    def _(i):
        idx = didx_v[pl.ds(i * L, L)]
        w = w_v[pl.ds(i * L, L)]
        plsc.addupdate_scatter(acc_v, [idx], w)

    pltpu.sync_copy(acc_v, out_hbm.at[wid])


@functools.partial(
    pl.kernel,
    out_type=jax.ShapeDtypeStruct((NC, PN, DH), _f32),
    mesh=_mesh,
    compiler_params=_sc_params,
    scratch_types=[
        pltpu.VMEM((KCH, CH), _i32),
        pltpu.VMEM((KCH, CH), _i32),
        pltpu.VMEM((KCH, CH), _f32),
        pltpu.VMEM((3, G, CH, DH), _f32),
        pltpu.VMEM_SHARED((PN, DH), _f32),
        pltpu.SemaphoreType.DMA,
        pltpu.SemaphoreType.DMA,
        pltpu.SemaphoreType.DMA,
        pltpu.SemaphoreType.DMA,
        pltpu.SemaphoreType.DMA,
        pltpu.SemaphoreType.DMA,
    ],
)
def _l1_kernel(y_hbm, src_hbm, dst_hbm, ew_hbm, out_hbm,
               sidxB, didxB, wB, rows_v, acc_sh,
               sg0, sg1, sg2, ss0, ss1, ss2):
    c = lax.axis_index("c")
    s = lax.axis_index("s")
    wid = c * NS + s
    semg = (sg0, sg1, sg2)
    sems = (ss0, ss1, ss2)
    rb = wid * KCH
    cp_s = pltpu.async_copy(src_hbm.at[pl.ds(rb, KCH)], sidxB, sg0)
    cp_d = pltpu.async_copy(dst_hbm.at[pl.ds(rb, KCH)], didxB, sg0)
    cp_w = pltpu.async_copy(ew_hbm.at[pl.ds(rb, KCH)], wB, sg0)
    zrow = jnp.zeros((L,), _f32)

    def zr(r, carry):
        for j in range(DH // L):
            rows_v[0, 0, r, pl.ds(j * L, L)] = zrow
        return carry

    lax.fori_loop(0, CH, zr, 0)
    rows_per_tile = PN // NS

    def zc(k, carry):
        pltpu.sync_copy(rows_v.at[0, 0],
                        acc_sh.at[pl.ds(s * rows_per_tile + k * CH, CH)])
        return carry

    lax.fori_loop(0, rows_per_tile // CH, zc, 0)
    plsc.subcore_barrier()
    cp_s.wait()
    cp_d.wait()
    cp_w.wait()

    def fire_gather(sk, b):
        for g in range(G):
            pltpu.async_copy(y_hbm.at[sidxB.at[sk * G + g]],
                             rows_v.at[b, g], semg[b])

    def wait_gather(sk, b):
        for g in range(G):
            pltpu.make_async_copy(y_hbm.at[sidxB.at[sk * G + g]],
                                  rows_v.at[b, g], semg[b]).wait()

    def fire_scatter(sk, b):
        for g in range(G):
            pltpu.async_copy(rows_v.at[b, g], acc_sh.at[didxB.at[sk * G + g]],
                             sems[b], add=True)

    def wait_scatter(sk, b):
        for g in range(G):
            pltpu.make_async_copy(rows_v.at[b, g],
                                  acc_sh.at[didxB.at[sk * G + g]],
                                  sems[b]).wait()

    fire_gather(0, 0)

    def slotf(sk, b):
        b1 = (b + 1) % 3

        @pl.when(sk < SK)
        def _():
            @pl.when(sk >= 2)
            def _():
                wait_scatter(sk - 2, b1)

            @pl.when(sk + 1 < SK)
            def _():
                fire_gather(sk + 1, b1)

            wait_gather(sk, b)

            @plsc.parallel_loop(0, CH, 1, unroll=4)
            def _(r):
                ridx = jnp.full((L,), r, _i32)
                for g in range(G):
                    w = plsc.load_gather(wB.at[sk * G + g], [ridx])
                    for j in range(DH // L):
                        rows_v[b, g, r, pl.ds(j * L, L)] = (
                            rows_v[b, g, r, pl.ds(j * L, L)] * w)

            fire_scatter(sk, b)

    def body(t, carry):
        for b in range(3):
            slotf(t * 3 + b, b)
        return carry

    lax.fori_loop(0, (SK + 2) // 3, body, 0)
    for j in range(SK - 2, SK):
        wait_scatter(j, j % 3)
    plsc.subcore_barrier()

    def xb(k, carry):
        r0 = s * rows_per_tile + k * CH
        pltpu.sync_copy(acc_sh.at[pl.ds(r0, CH)], out_hbm.at[c, pl.ds(r0, CH)])
        return carry

    lax.fori_loop(0, rows_per_tile // CH, xb, 0)


@functools.partial(
    pl.kernel,
    out_type=jax.ShapeDtypeStruct((NW, PN), _f32),
    mesh=_mesh,
    compiler_params=_sc_params,
    scratch_types=[
        pltpu.VMEM((EPW,), _i32),
        pltpu.VMEM((EPW,), _i32),
        pltpu.VMEM((EPW,), _f32),
        pltpu.VMEM((N,), _f32),
        pltpu.VMEM((PN,), _f32),
    ],
)
def _l2_kernel(src_hbm, dst_hbm, ew_hbm, y2_hbm, out_hbm,
               sidx_v, didx_v, w_v, tab_v, acc_v):
    c = lax.axis_index("c")
    s = lax.axis_index("s")
    wid = c * NS + s
    _zero_1d(acc_v, PN)
    base = wid * EPW
    pltpu.sync_copy(src_hbm.at[pl.ds(base, EPW)], sidx_v)
    pltpu.sync_copy(dst_hbm.at[pl.ds(base, EPW)], didx_v)
    pltpu.sync_copy(ew_hbm.at[pl.ds(base, EPW)], w_v)
    pltpu.sync_copy(y2_hbm, tab_v)

    @plsc.parallel_loop(0, EPW // L, 1, unroll=8)
    def _(i):
        s16 = sidx_v[pl.ds(i * L, L)]
        d16 = didx_v[pl.ds(i * L, L)]
        w16 = w_v[pl.ds(i * L, L)]
        vals = plsc.load_gather(tab_v, [s16])
        plsc.addupdate_scatter(acc_v, [d16], w16 * vals)

    pltpu.sync_copy(acc_v, out_hbm.at[wid])


def _lrelu(z):
    return jnp.where(z >= 0, z, 0.01 * z)


def _xw_body(x_ref, w1_ref, xw_ref):
    xw_ref[...] = jnp.dot(x_ref[...], w1_ref[...],
                          preferred_element_type=_f32)


def _prep_body(degp_ref, xw_ref, y_ref, dinv_ref):
    deg = 1.0 + jnp.sum(degp_ref[...], axis=1, keepdims=True)
    dinv = jnp.where(deg > 0, lax.rsqrt(deg), 0.0)
    d = dinv[:N]
    y_ref[...] = xw_ref[...] * d
    dinv_ref[...] = d


def _mid_body(accp_ref, y_ref, dinv_ref, b1_ref, w2_ref, y2_ref):
    acc = accp_ref[0, :N, :] + accp_ref[1, :N, :]
    d = dinv_ref[...]
    h1 = _lrelu(d * (acc + y_ref[...]) + b1_ref[...])
    hw = jnp.dot(h1, w2_ref[...], preferred_element_type=_f32)
    y2_ref[...] = d * hw


def _fin_body(acc2p_ref, y2_ref, dinv_ref, b2_ref, fc1w_ref, fc1b_ref,
              fc2w_ref, fc2b_ref, out_ref):
    a2 = jnp.sum(acc2p_ref[...], axis=1, keepdims=True)
    h2 = _lrelu(dinv_ref[...] * (a2 + y2_ref[...]) + b2_ref[...])
    t = jnp.dot(fc1w_ref[...], h2, preferred_element_type=_f32)
    t = _lrelu(t + fc1b_ref[...])
    logits = jnp.dot(fc2w_ref[...], t, preferred_element_type=_f32)
    logits = logits + fc2b_ref[...]
    m = jnp.max(logits, axis=0, keepdims=True)
    e = jnp.exp(logits - m)
    out_ref[...] = e / jnp.sum(e, axis=0, keepdims=True)


def kernel(x, edge_index, edge_attr, W1, b1, W2, b2, fc1_w, fc1_b, fc2_w, fc2_b):
    src = edge_index[0]
    dst = edge_index[1]
    ew = edge_attr[:, 0]

    degp = _deg_kernel(dst, ew)
    xw = pl.pallas_call(
        _xw_body,
        out_shape=jax.ShapeDtypeStruct((N, DH), _f32),
    )(x, W1)
    y, dinv = pl.pallas_call(
        _prep_body,
        out_shape=[
            jax.ShapeDtypeStruct((N, DH), _f32),
            jax.ShapeDtypeStruct((N, 1), _f32),
        ],
    )(degp.T, xw)

    pad = EP - E
    pidx = jnp.arange(pad, dtype=jnp.int32)
    src_p = jnp.concatenate([src, pidx % N]).reshape(EP // CH, CH)
    dst_p = jnp.concatenate([dst, N + pidx % (PN - N)]).reshape(EP // CH, CH)
    ew_p = jnp.concatenate(
        [ew, jnp.zeros((pad,), _f32)]).reshape(EP // CH, CH)
    accp = _l1_kernel(y, src_p, dst_p, ew_p)
    y2 = pl.pallas_call(
        _mid_body,
        out_shape=jax.ShapeDtypeStruct((N, 1), _f32),
    )(accp, y, dinv, b1.reshape(1, DH), W2)

    acc2p = _l2_kernel(src, dst, ew, y2.reshape(N))
    out = pl.pallas_call(
        _fin_body,
        out_shape=jax.ShapeDtypeStruct((OUT, 1), _f32),
    )(acc2p[:, :N].T, y2, dinv, b2.reshape(1, 1), fc1_w, fc1_b.reshape(D_IN, 1),
      fc2_w, fc2_b.reshape(OUT, 1))
    return out.reshape(1, OUT)
